# Initial kernel scaffold; baseline (speedup 1.0000x reference)
import jax
import jax.numpy as jnp
from jax.experimental import pallas as pl


def kernel(stamp, table, ln_gamma, ln_beta):
    raise NotImplementedError("write your pallas kernel here")


# speedup vs baseline: 3.9240x; 3.9240x over previous
import numpy as np
import jax
import jax.numpy as jnp
from jax import lax
from jax.experimental import pallas as pl
from jax.experimental.pallas import tpu as pltpu
from jax.experimental.pallas import tpu_sc as plsc

_B, _L, _D, _V = 16384, 200, 16, 100000
_N = _B * _L
_NC, _NS = 2, 16
_NW = _NC * _NS
_TPW = _N // _NW
_CHUNK = 800
_NCH = _TPW // _CHUNK
_NG = _CHUNK // 16
_PHASES = 25


def _pe_table():
    pos = np.arange(_L, dtype=np.float32)[:, None]
    i = np.arange(_D, dtype=np.float32)[None, :]
    angle = pos / np.power(10000.0, (2.0 * np.floor(i / 2.0)) / _D)
    pe = np.zeros((_L, _D), dtype=np.float32)
    pe[:, 0::2] = np.sin(angle[:, 0::2])
    pe[:, 1::2] = np.cos(angle[:, 1::2])
    return pe


def _pe_phases():
    pe = _pe_table()
    ph = np.arange(_PHASES)[:, None]
    lane = np.arange(16)[None, :]
    pos = (ph * 16 + lane) % _L
    tab = pe[pos]
    return np.ascontiguousarray(tab.transpose(0, 2, 1)).reshape(-1)


_PE_PHASE = _pe_phases()


def _rsqrt(a):
    bits = lax.bitcast_convert_type(a, jnp.int32)
    y = lax.bitcast_convert_type(
        jnp.int32(0x5F3759DF) - lax.shift_right_arithmetic(bits, 1),
        jnp.float32)
    for _ in range(3):
        y = y * (1.5 - 0.5 * a * y * y)
    return y


def _tec_body(stamp_hbm, pe_hbm, table_hbm, gamma_hbm, beta_hbm, out_hbm,
              idx_v, rows_v, out_v, pe_v, g_v, b_v, sem):
    wid = lax.axis_index("s") * _NC + lax.axis_index("c")
    pltpu.sync_copy(pe_hbm, pe_v)
    pltpu.sync_copy(gamma_hbm, g_v)
    pltpu.sync_copy(beta_hbm, b_v)
    base0 = wid * _TPW

    def group_body(g, carry):
        iota = lax.iota(jnp.int32, 16)
        rowidx = g * 16 + iota
        pe_off = lax.rem(g, _PHASES) * (_D * 16)
        s1 = jnp.zeros((16,), jnp.float32)
        xs = []
        for d in range(_D):
            dfull = jnp.full((16,), d, jnp.int32)
            e = plsc.load_gather(rows_v, [rowidx, dfull])
            p = pe_v[pl.ds(pe_off + d * 16, 16)]
            x = e + p
            s1 = s1 + x
            xs.append(x)
        mu = s1 * (1.0 / _D)
        s2 = jnp.zeros((16,), jnp.float32)
        for d in range(_D):
            dev = xs[d] - mu
            s2 = s2 + dev * dev
            xs[d] = dev
        r = _rsqrt(s2 * (1.0 / _D) + 1e-5)
        for d in range(_D):
            dfull = jnp.full((16,), d, jnp.int32)
            gs = plsc.load_gather(g_v, [dfull])
            bs = plsc.load_gather(b_v, [dfull])
            plsc.store_scatter(out_v, [rowidx, dfull], xs[d] * r * gs + bs)
        return carry

    def chunk_body(c, carry):
        tb = base0 + c * _CHUNK
        pltpu.sync_copy(stamp_hbm.at[pl.ds(tb, _CHUNK)], idx_v)
        pltpu.async_copy(table_hbm.at[idx_v], rows_v, sem).wait()
        lax.fori_loop(0, _NG, group_body, 0)
        pltpu.sync_copy(out_v, out_hbm.at[pl.ds(tb, _CHUNK)])
        return carry

    lax.fori_loop(0, _NCH, chunk_body, 0)


def kernel(stamp, table, ln_gamma, ln_beta):
    stamp_flat = stamp.reshape(-1)
    pe_const = jnp.asarray(_PE_PHASE)
    mesh = plsc.VectorSubcoreMesh(core_axis_name="c", subcore_axis_name="s")
    run = pl.kernel(
        _tec_body,
        compiler_params=pltpu.CompilerParams(
            needs_layout_passes=False, use_tc_tiling_on_sc=False),
        out_type=jax.ShapeDtypeStruct((_N, _D), jnp.float32),
        mesh=mesh,
        scratch_types=[
            pltpu.VMEM((_CHUNK,), jnp.int32),
            pltpu.VMEM((_CHUNK, _D), jnp.float32),
            pltpu.VMEM((_CHUNK, _D), jnp.float32),
            pltpu.VMEM((_PHASES * _D * 16,), jnp.float32),
            pltpu.VMEM((_D,), jnp.float32),
            pltpu.VMEM((_D,), jnp.float32),
            pltpu.SemaphoreType.DMA,
        ],
    )
    out = run(stamp_flat, pe_const, table, ln_gamma, ln_beta)
    return out.reshape(_B, _L, _D)


# speedup vs baseline: 5.0662x; 1.2911x over previous
import numpy as np
import jax
import jax.numpy as jnp
from jax import lax
from jax.experimental import pallas as pl
from jax.experimental.pallas import tpu as pltpu
from jax.experimental.pallas import tpu_sc as plsc

_B, _L, _D, _V = 16384, 200, 16, 100000
_N = _B * _L
_NC, _NS = 2, 16
_NW = _NC * _NS
_TPW = _N // _NW
_CHUNK = 800
_NCH = _TPW // _CHUNK
_NG = _CHUNK // 16
_PHASES = 25


def _pe_table():
    pos = np.arange(_L, dtype=np.float32)[:, None]
    i = np.arange(_D, dtype=np.float32)[None, :]
    angle = pos / np.power(10000.0, (2.0 * np.floor(i / 2.0)) / _D)
    pe = np.zeros((_L, _D), dtype=np.float32)
    pe[:, 0::2] = np.sin(angle[:, 0::2])
    pe[:, 1::2] = np.cos(angle[:, 1::2])
    return pe


def _pe_phases():
    pe = _pe_table()
    ph = np.arange(_PHASES)[:, None]
    lane = np.arange(16)[None, :]
    pos = (ph * 16 + lane) % _L
    tab = pe[pos]
    return np.ascontiguousarray(tab.transpose(0, 2, 1)).reshape(-1)


_PE_PHASE = _pe_phases()


def _rsqrt(a):
    bits = lax.bitcast_convert_type(a, jnp.int32)
    y = lax.bitcast_convert_type(
        jnp.int32(0x5F3759DF) - lax.shift_right_arithmetic(bits, 1),
        jnp.float32)
    for _ in range(3):
        y = y * (1.5 - 0.5 * a * y * y)
    return y


def _tec_body(stamp_hbm, pe_hbm, table_hbm, gamma_hbm, beta_hbm, out_hbm,
              idx_v, rows_v, out_v, pe_v, g_v, b_v, isem, gsem, osem):
    wid = lax.axis_index("s") * _NC + lax.axis_index("c")
    pltpu.sync_copy(pe_hbm, pe_v)
    pltpu.sync_copy(gamma_hbm, g_v)
    pltpu.sync_copy(beta_hbm, b_v)
    base0 = wid * _TPW
    iota = lax.iota(jnp.int32, 16)
    iota16 = iota * 16
    gsp = [plsc.load_gather(g_v, [jnp.full((16,), d, jnp.int32)])
           for d in range(_D)]
    bsp = [plsc.load_gather(b_v, [jnp.full((16,), d, jnp.int32)])
           for d in range(_D)]

    def start_idx(c, b):
        tb = base0 + c * _CHUNK
        pltpu.async_copy(stamp_hbm.at[pl.ds(tb, _CHUNK)], idx_v[b], isem[b])

    def wait_idx(b):
        pltpu.make_async_copy(stamp_hbm.at[pl.ds(0, _CHUNK)], idx_v[b],
                              isem[b]).wait()

    def start_gather(b):
        pltpu.async_copy(table_hbm.at[idx_v[b]], rows_v[b], gsem[b])

    def wait_gather(b):
        pltpu.make_async_copy(table_hbm.at[idx_v[b]], rows_v[b],
                              gsem[b]).wait()

    def start_out(c, b):
        tb16 = (base0 + c * _CHUNK) * _D
        pltpu.async_copy(out_v[b], out_hbm.at[pl.ds(tb16, _CHUNK * _D)],
                         osem[b])

    def wait_out(b):
        pltpu.make_async_copy(out_v[b], out_hbm.at[pl.ds(0, _CHUNK * _D)],
                              osem[b]).wait()

    def compute(rows, out):
        def group_body(g, carry):
            base_vec = g * 256 + iota16
            rowidx = g * 16 + iota
            pe_off = lax.rem(g, _PHASES) * (_D * 16)
            s1 = jnp.zeros((16,), jnp.float32)
            xs = []
            for d in range(_D):
                e = plsc.load_gather(rows, [rowidx,
                                            jnp.full((16,), d, jnp.int32)])
                x = e + pe_v[pl.ds(pe_off + d * 16, 16)]
                s1 = s1 + x
                xs.append(x)
            mu = s1 * (1.0 / _D)
            s2 = jnp.zeros((16,), jnp.float32)
            for d in range(_D):
                dev = xs[d] - mu
                s2 = s2 + dev * dev
                xs[d] = dev
            r = _rsqrt(s2 * (1.0 / _D) + 1e-5)
            for d in range(_D):
                plsc.store_scatter(out, [base_vec + d],
                                   xs[d] * (r * gsp[d]) + bsp[d])
            return carry
        lax.fori_loop(0, _NG, group_body, 0)

    start_idx(0, 0)
    start_idx(1, 1)
    wait_idx(0)
    start_gather(0)

    def outer(i, carry):
        for b in range(2):
            c = 2 * i + b

            @pl.when(c + 1 < _NCH)
            def _():
                wait_idx(b ^ 1)
                start_gather(b ^ 1)

            wait_gather(b)

            @pl.when(c >= 2)
            def _():
                wait_out(b)

            compute(rows_v[b], out_v[b])
            start_out(c, b)

            @pl.when(c + 2 < _NCH)
            def _():
                start_idx(c + 2, b)
        return carry

    lax.fori_loop(0, _NCH // 2, outer, 0)
    wait_out(0)
    wait_out(1)


def kernel(stamp, table, ln_gamma, ln_beta):
    stamp_flat = stamp.reshape(-1)
    pe_const = jnp.asarray(_PE_PHASE)
    mesh = plsc.VectorSubcoreMesh(core_axis_name="c", subcore_axis_name="s")
    run = pl.kernel(
        _tec_body,
        compiler_params=pltpu.CompilerParams(
            needs_layout_passes=False, use_tc_tiling_on_sc=False),
        out_type=jax.ShapeDtypeStruct((_N * _D,), jnp.float32),
        mesh=mesh,
        scratch_types=[
            [pltpu.VMEM((_CHUNK,), jnp.int32)] * 2,
            [pltpu.VMEM((_CHUNK, _D), jnp.float32)] * 2,
            [pltpu.VMEM((_CHUNK * _D,), jnp.float32)] * 2,
            pltpu.VMEM((_PHASES * _D * 16,), jnp.float32),
            pltpu.VMEM((_D,), jnp.float32),
            pltpu.VMEM((_D,), jnp.float32),
            [pltpu.SemaphoreType.DMA] * 2,
            [pltpu.SemaphoreType.DMA] * 2,
            [pltpu.SemaphoreType.DMA] * 2,
        ],
    )
    out = run(stamp_flat, pe_const, table, ln_gamma, ln_beta)
    return out.reshape(_B, _L, _D)


# speedup vs baseline: 14.6681x; 2.8953x over previous
import numpy as np
import jax
import jax.numpy as jnp
from jax import lax
from jax.experimental import pallas as pl
from jax.experimental.pallas import tpu as pltpu
from jax.experimental.pallas import tpu_sc as plsc

_B, _L, _D, _V = 16384, 200, 16, 100000
_N = _B * _L
_NC, _NS = 2, 16
_NW = _NC * _NS
_BPW = _B // _NW
_NG = _BPW // 16


def _pe_lane_table():
    pos = np.arange(_L, dtype=np.float32)[:, None]
    i = np.arange(_D, dtype=np.float32)[None, :]
    angle = pos / np.power(10000.0, (2.0 * np.floor(i / 2.0)) / _D)
    pe = np.zeros((_L, _D), dtype=np.float32)
    pe[:, 0::2] = np.sin(angle[:, 0::2])
    pe[:, 1::2] = np.cos(angle[:, 1::2])
    return np.ascontiguousarray(
        np.broadcast_to(pe[:, :, None], (_L, _D, 16))).reshape(-1)


_PE_LANE = _pe_lane_table()


def _rsqrt(a):
    bits = lax.bitcast_convert_type(a, jnp.int32)
    y = lax.bitcast_convert_type(
        jnp.int32(0x5F3759DF) - lax.shift_right_arithmetic(bits, 1),
        jnp.float32)
    for _ in range(3):
        y = y * (1.5 - 0.5 * a * y * y)
    return y


def _tec_body(stamp_hbm, pe_hbm, table_hbm, gamma_hbm, beta_hbm, out_hbm,
              idx_v, rows_v, out_v, pe_v, g_v, b_v, isem, gsem, osem):
    wid = lax.axis_index("s") * _NC + lax.axis_index("c")
    pltpu.sync_copy(pe_hbm, pe_v)
    pltpu.sync_copy(gamma_hbm, g_v)
    pltpu.sync_copy(beta_hbm, b_v)
    b0 = wid * _BPW
    iota = lax.iota(jnp.int32, 16)
    gsp = [plsc.load_gather(g_v, [jnp.full((16,), d, jnp.int32)])
           for d in range(_D)]
    bsp = [plsc.load_gather(b_v, [jnp.full((16,), d, jnp.int32)])
           for d in range(_D)]

    def start_idx(l, b):
        off = l * _B + b0
        pltpu.async_copy(stamp_hbm.at[pl.ds(off, _BPW)], idx_v[b], isem[b])

    def wait_idx(b):
        pltpu.make_async_copy(stamp_hbm.at[pl.ds(0, _BPW)], idx_v[b],
                              isem[b]).wait()

    def start_gather(b):
        pltpu.async_copy(table_hbm.at[idx_v[b]], rows_v[b], gsem[b])

    def wait_gather(b):
        pltpu.make_async_copy(table_hbm.at[idx_v[b]], rows_v[b],
                              gsem[b]).wait()

    def start_out(l, b):
        for i in range(2):
            off = (l * 256 + i * 128 + wid * 4) * 1024
            pltpu.async_copy(out_v[b].at[pl.ds(i * 4096, 4096)],
                             out_hbm.at[pl.ds(off, 4096)], osem[b])

    def wait_out(b):
        for i in range(2):
            pltpu.make_async_copy(out_v[b].at[pl.ds(i * 4096, 4096)],
                                  out_hbm.at[pl.ds(0, 4096)], osem[b]).wait()

    def compute(l, rows, out):
        lbase = l * 256

        def group_body(g, carry):
            rowidx = g * 16 + iota
            gpart = (g // 8) * 1024 + (g % 8) * 16
            s1 = jnp.zeros((16,), jnp.float32)
            xs = []
            for d in range(_D):
                e = plsc.load_gather(rows, [rowidx,
                                            jnp.full((16,), d, jnp.int32)])
                x = e + pe_v[pl.ds(lbase + d * 16, 16)]
                s1 = s1 + x
                xs.append(x)
            mu = s1 * (1.0 / _D)
            s2 = jnp.zeros((16,), jnp.float32)
            for d in range(_D):
                dev = xs[d] - mu
                s2 = s2 + dev * dev
                xs[d] = dev
            r = _rsqrt(s2 * (1.0 / _D) + 1e-5)
            for d in range(_D):
                obase = gpart + (d // 8) * 4096 + (d % 8) * 128
                plsc.store_scatter(out, [obase + iota],
                                   xs[d] * (r * gsp[d]) + bsp[d])
            return carry

        lax.fori_loop(0, _NG, group_body, 0)

    start_idx(0, 0)
    start_idx(1, 1)
    wait_idx(0)
    start_gather(0)

    def outer(i, carry):
        for b in range(2):
            c = 2 * i + b

            @pl.when(c + 1 < _L)
            def _():
                wait_idx(b ^ 1)
                start_gather(b ^ 1)

            wait_gather(b)

            @pl.when(c >= 2)
            def _():
                wait_out(b)

            compute(c, rows_v[b], out_v[b])
            start_out(c, b)

            @pl.when(c + 2 < _L)
            def _():
                start_idx(c + 2, b)
        return carry

    lax.fori_loop(0, _L // 2, outer, 0)
    wait_out(0)
    wait_out(1)


def kernel(stamp, table, ln_gamma, ln_beta):
    stamp_t = jnp.transpose(stamp).reshape(-1)
    pe_const = jnp.asarray(_PE_LANE)
    mesh = plsc.VectorSubcoreMesh(core_axis_name="c", subcore_axis_name="s")
    run = pl.kernel(
        _tec_body,
        compiler_params=pltpu.CompilerParams(
            needs_layout_passes=False, use_tc_tiling_on_sc=False),
        out_type=jax.ShapeDtypeStruct((_N * _D,), jnp.float32),
        mesh=mesh,
        scratch_types=[
            [pltpu.VMEM((_BPW,), jnp.int32)] * 2,
            [pltpu.VMEM((_BPW, _D), jnp.float32)] * 2,
            [pltpu.VMEM((_BPW * _D,), jnp.float32)] * 2,
            pltpu.VMEM((_L * _D * 16,), jnp.float32),
            pltpu.VMEM((_D,), jnp.float32),
            pltpu.VMEM((_D,), jnp.float32),
            [pltpu.SemaphoreType.DMA] * 2,
            [pltpu.SemaphoreType.DMA] * 2,
            [pltpu.SemaphoreType.DMA] * 2,
        ],
    )
    out = run(stamp_t, pe_const, table, ln_gamma, ln_beta)
    out = out.reshape(_L, 2, _B // 128, 8, 128)
    return jnp.transpose(out, (2, 4, 0, 1, 3)).reshape(_B, _L, _D)


# speedup vs baseline: 17.2848x; 1.1784x over previous
import numpy as np
import jax
import jax.numpy as jnp
from jax import lax
from jax.experimental import pallas as pl
from jax.experimental.pallas import tpu as pltpu
from jax.experimental.pallas import tpu_sc as plsc

_B, _L, _D, _V = 16384, 200, 16, 100000
_N = _B * _L
_NC, _NS = 2, 16
_NW = _NC * _NS
_BPW = _B // _NW
_NG = _BPW // 16


def _pe_lane_table():
    pos = np.arange(_L, dtype=np.float32)[:, None]
    i = np.arange(_D, dtype=np.float32)[None, :]
    angle = pos / np.power(10000.0, (2.0 * np.floor(i / 2.0)) / _D)
    pe = np.zeros((_L, _D), dtype=np.float32)
    pe[:, 0::2] = np.sin(angle[:, 0::2])
    pe[:, 1::2] = np.cos(angle[:, 1::2])
    return np.ascontiguousarray(
        np.broadcast_to(pe[:, :, None], (_L, _D, 16))).reshape(-1)


_PE_LANE = _pe_lane_table()


def _rsqrt(a):
    bits = lax.bitcast_convert_type(a, jnp.int32)
    y = lax.bitcast_convert_type(
        jnp.int32(0x5F3759DF) - lax.shift_right_arithmetic(bits, 1),
        jnp.float32)
    for _ in range(2):
        y = y * (1.5 - 0.5 * a * y * y)
    return y


def _tree_sum(vs):
    vs = list(vs)
    while len(vs) > 1:
        nxt = [vs[i] + vs[i + 1] for i in range(0, len(vs) - 1, 2)]
        if len(vs) % 2:
            nxt.append(vs[-1])
        vs = nxt
    return vs[0]


def _tec_body(stamp_hbm, pe_hbm, table_hbm, gamma_hbm, beta_hbm, out_hbm,
              idx_v, rows_v, out_v, pe_v, g_v, b_v, isem, gsem, osem):
    wid = lax.axis_index("s") * _NC + lax.axis_index("c")
    pltpu.sync_copy(pe_hbm, pe_v)
    pltpu.sync_copy(gamma_hbm, g_v)
    pltpu.sync_copy(beta_hbm, b_v)
    b0 = wid * _BPW
    iota = lax.iota(jnp.int32, 16)
    gsp = [plsc.load_gather(g_v, [jnp.full((16,), d, jnp.int32)])
           for d in range(_D)]
    bsp = [plsc.load_gather(b_v, [jnp.full((16,), d, jnp.int32)])
           for d in range(_D)]

    def start_idx(l, b):
        off = l * _B + b0
        pltpu.async_copy(stamp_hbm.at[pl.ds(off, _BPW)], idx_v[b], isem[b])

    def wait_idx(b):
        pltpu.make_async_copy(stamp_hbm.at[pl.ds(0, _BPW)], idx_v[b],
                              isem[b]).wait()

    def start_gather(b):
        pltpu.async_copy(table_hbm.at[idx_v[b]], rows_v[b], gsem[b])

    def wait_gather(b):
        pltpu.make_async_copy(table_hbm.at[idx_v[b]], rows_v[b],
                              gsem[b]).wait()

    def start_out(l, b):
        for i in range(2):
            off = (l * 256 + i * 128 + wid * 4) * 1024
            pltpu.async_copy(out_v[b].at[pl.ds(i * 4096, 4096)],
                             out_hbm.at[pl.ds(off, 4096)], osem[b])

    def wait_out(b):
        for i in range(2):
            pltpu.make_async_copy(out_v[b].at[pl.ds(i * 4096, 4096)],
                                  out_hbm.at[pl.ds(0, 4096)], osem[b]).wait()

    def compute(l, rows, out):
        lbase = l * 256
        psp = [pe_v[pl.ds(lbase + d * 16, 16)] for d in range(_D)]

        def one_group(g):
            rowidx = g * 16 + iota
            gpart = (g // 8) * 1024 + (g % 8) * 16
            xs = []
            for d in range(_D):
                e = plsc.load_gather(rows, [rowidx,
                                            jnp.full((16,), d, jnp.int32)])
                xs.append(e + psp[d])
            mu = _tree_sum(xs) * (1.0 / _D)
            devs = [x - mu for x in xs]
            s2 = _tree_sum([dv * dv for dv in devs])
            r = _rsqrt(s2 * (1.0 / _D) + 1e-5)
            for d in range(_D):
                obase = gpart + (d // 8) * 4096 + (d % 8) * 128
                plsc.store_scatter(out, [obase + iota],
                                   devs[d] * (r * gsp[d]) + bsp[d])

        def group_body(h, carry):
            one_group(2 * h)
            one_group(2 * h + 1)
            return carry

        lax.fori_loop(0, _NG // 2, group_body, 0)

    start_idx(0, 0)
    start_idx(1, 1)
    wait_idx(0)
    start_gather(0)

    def outer(i, carry):
        for b in range(2):
            c = 2 * i + b

            @pl.when(c + 1 < _L)
            def _():
                wait_idx(b ^ 1)
                start_gather(b ^ 1)

            wait_gather(b)

            @pl.when(c >= 2)
            def _():
                wait_out(b)

            compute(c, rows_v[b], out_v[b])
            start_out(c, b)

            @pl.when(c + 2 < _L)
            def _():
                start_idx(c + 2, b)
        return carry

    lax.fori_loop(0, _L // 2, outer, 0)
    wait_out(0)
    wait_out(1)


def kernel(stamp, table, ln_gamma, ln_beta):
    stamp_t = jnp.transpose(stamp).reshape(-1)
    pe_const = jnp.asarray(_PE_LANE)
    mesh = plsc.VectorSubcoreMesh(core_axis_name="c", subcore_axis_name="s")
    run = pl.kernel(
        _tec_body,
        compiler_params=pltpu.CompilerParams(
            needs_layout_passes=False, use_tc_tiling_on_sc=False),
        out_type=jax.ShapeDtypeStruct((_N * _D,), jnp.float32),
        mesh=mesh,
        scratch_types=[
            [pltpu.VMEM((_BPW,), jnp.int32)] * 2,
            [pltpu.VMEM((_BPW, _D), jnp.float32)] * 2,
            [pltpu.VMEM((_BPW * _D,), jnp.float32)] * 2,
            pltpu.VMEM((_L * _D * 16,), jnp.float32),
            pltpu.VMEM((_D,), jnp.float32),
            pltpu.VMEM((_D,), jnp.float32),
            [pltpu.SemaphoreType.DMA] * 2,
            [pltpu.SemaphoreType.DMA] * 2,
            [pltpu.SemaphoreType.DMA] * 2,
        ],
    )
    out = run(stamp_t, pe_const, table, ln_gamma, ln_beta)
    out = out.reshape(_L, 2, _B // 128, 8, 128)
    return jnp.transpose(out, (2, 4, 0, 1, 3)).reshape(_B, _L, _D)
